# baseline (speedup 1.0000x reference)
import functools

import jax
import jax.numpy as jnp
from jax import lax
from jax.experimental import pallas as pl
from jax.experimental.pallas import tpu as pltpu
from jax.experimental.pallas import tpu_sc as plsc

NC = 2
NS = 16
NW = NC * NS

SUB = 128
K = 8
CHUNK = K * SUB


def _make_emb_kernel(B, D, DP):
    assert B % (NW * CHUNK) == 0
    b_per_w = B // NW
    n_chunks = b_per_w // CHUNK

    mesh = plsc.VectorSubcoreMesh(core_axis_name="c", subcore_axis_name="s")

    @functools.partial(
        pl.kernel,
        mesh=mesh,
        out_type=jax.ShapeDtypeStruct((B * D,), jnp.float32),
        scratch_types=(
            [pltpu.VMEM((SUB,), jnp.int32) for _ in range(K)]
            + [
                pltpu.VMEM((CHUNK, DP), jnp.float32),
                pltpu.VMEM((CHUNK * D,), jnp.float32),
                pltpu.SemaphoreType.DMA,
                pltpu.SemaphoreType.DMA,
            ]
        ),
        compiler_params=pltpu.CompilerParams(use_tc_tiling_on_sc=False),
    )
    def emb(idx_hbm, table_hbm, out_hbm, *rest):
        idx_bufs = rest[:K]
        rows_v = rest[K]
        dense_v = rest[K + 1]
        sem_i = rest[K + 2]
        sem_g = rest[K + 3]
        wid = lax.axis_index("s") * NC + lax.axis_index("c")
        base = wid * b_per_w

        def chunk_body(i, _):
            off = base + i * CHUNK
            icopies = [
                pltpu.async_copy(
                    idx_hbm.at[pl.ds(off + j * SUB, SUB)], idx_bufs[j], sem_i
                )
                for j in range(K)
            ]
            for c in icopies:
                c.wait()
            gcopies = [
                pltpu.async_copy(
                    table_hbm.at[idx_bufs[j]],
                    rows_v.at[pl.ds(j * SUB, SUB)],
                    sem_g,
                )
                for j in range(K)
            ]
            for c in gcopies:
                c.wait()

            def row_body(r, _):
                d = r * D
                for o in (0, 16, 32, D - 16):
                    dense_v[pl.ds(d + o, 16)] = rows_v[r, pl.ds(o, 16)]
                return ()

            lax.fori_loop(0, CHUNK, row_body, (), unroll=4)
            pltpu.sync_copy(dense_v, out_hbm.at[pl.ds(off * D, CHUNK * D)])
            return ()

        lax.fori_loop(0, n_chunks, chunk_body, ())

    return emb


def kernel(indices, table):
    BATCH, HIST = indices.shape
    V, D = table.shape
    DP = (D + 7) // 8 * 8
    B = BATCH * HIST
    idx_flat = indices.reshape(B).astype(jnp.int32)
    table_p = jnp.pad(table, ((0, 0), (0, DP - D)))
    emb = _make_emb_kernel(B, D, DP)
    out = emb(idx_flat, table_p)
    return out.reshape(BATCH, HIST, D)


# speedup vs baseline: 1.0674x; 1.0674x over previous
import functools

import jax
import jax.numpy as jnp
from jax import lax
from jax.experimental import pallas as pl
from jax.experimental.pallas import tpu as pltpu
from jax.experimental.pallas import tpu_sc as plsc

NC = 2
NS = 16
NW = NC * NS

SUB = 128
K = 4
CHUNK = K * SUB
RB = 8


def _make_emb_kernel(B, D, DP):
    assert B % (NW * CHUNK) == 0
    b_per_w = B // NW
    n_chunks = b_per_w // CHUNK
    assert n_chunks % 2 == 0
    n_rows = B // SUB

    mesh = plsc.VectorSubcoreMesh(core_axis_name="c", subcore_axis_name="s")

    @functools.partial(
        pl.kernel,
        mesh=mesh,
        out_type=jax.ShapeDtypeStruct((B * D,), jnp.float32),
        scratch_types=[
            pltpu.VMEM((K, SUB), jnp.int32),
            pltpu.VMEM((K, SUB), jnp.int32),
            pltpu.VMEM((CHUNK, DP), jnp.float32),
            pltpu.VMEM((CHUNK, DP), jnp.float32),
            pltpu.VMEM((CHUNK * D,), jnp.float32),
            pltpu.SemaphoreType.DMA,
            pltpu.SemaphoreType.DMA,
            pltpu.SemaphoreType.DMA,
            pltpu.SemaphoreType.DMA,
        ],
        compiler_params=pltpu.CompilerParams(use_tc_tiling_on_sc=False),
    )
    def emb(idx_hbm, table_hbm, out_hbm, i0, i1, r0, r1, dense_v, si0, si1, sg0, sg1):
        idx_bufs = (i0, i1)
        rows_bufs = (r0, r1)
        sem_i = (si0, si1)
        sem_g = (sg0, sg1)
        wid = lax.axis_index("s") * NC + lax.axis_index("c")
        row_base = wid * (b_per_w // SUB)

        def fire_idx(c, buf):
            pltpu.async_copy(
                idx_hbm.at[pl.ds(row_base + c * K, K)], idx_bufs[buf], sem_i[buf]
            )

        def wait_idx(buf):
            pltpu.make_async_copy(
                idx_hbm.at[pl.ds(0, K)], idx_bufs[buf], sem_i[buf]
            ).wait()

        def fire_gathers(buf):
            for j in range(K):
                pltpu.async_copy(
                    table_hbm.at[idx_bufs[buf].at[j]],
                    rows_bufs[buf].at[pl.ds(j * SUB, SUB)],
                    sem_g[buf],
                )

        def wait_gathers(buf):
            pltpu.make_async_copy(
                table_hbm.at[pl.ds(0, CHUNK)], rows_bufs[buf], sem_g[buf]
            ).wait()

        def compact(buf):
            rows_v = rows_bufs[buf]

            def blk_body(t, carry):
                row0, d0 = carry
                for rr in range(RB):
                    for o in (0, 16, 32, D - 16):
                        dense_v[pl.ds(d0 + rr * D + o, 16)] = rows_v[
                            row0 + rr, pl.ds(o, 16)
                        ]
                return (row0 + RB, d0 + RB * D)

            lax.fori_loop(0, CHUNK // RB, blk_body, (0, 0))

        def phase(i, c, cur, nxt):
            n_more = n_chunks - 1

            @pl.when(c < n_more)
            def _():
                wait_idx(nxt)
                fire_gathers(nxt)

            wait_gathers(cur)
            compact(cur)

            @pl.when(c + 2 <= n_more)
            def _():
                fire_idx(c + 2, cur)

            off = (row_base + c * K) * SUB
            pltpu.sync_copy(dense_v, out_hbm.at[pl.ds(off * D, CHUNK * D)])

        fire_idx(0, 0)
        wait_idx(0)
        fire_gathers(0)
        fire_idx(1, 1)

        def pair_body(t, _):
            c = t * 2
            phase(t, c, 0, 1)
            phase(t, c + 1, 1, 0)
            return ()

        lax.fori_loop(0, n_chunks // 2, pair_body, ())

    return emb


def kernel(indices, table):
    BATCH, HIST = indices.shape
    V, D = table.shape
    DP = (D + 7) // 8 * 8
    B = BATCH * HIST
    idx2d = indices.reshape(B // SUB, SUB).astype(jnp.int32)
    table_p = jnp.pad(table, ((0, 0), (0, DP - D)))
    emb = _make_emb_kernel(B, D, DP)
    out = emb(idx2d, table_p)
    return out.reshape(BATCH, HIST, D)


# speedup vs baseline: 1.9181x; 1.7970x over previous
import functools

import jax
import jax.numpy as jnp
from jax import lax
from jax.experimental import pallas as pl
from jax.experimental.pallas import tpu as pltpu
from jax.experimental.pallas import tpu_sc as plsc

NC = 2
NS = 16
NW = NC * NS

SCB = 16
S1, S2 = 104, 96


def _make_emb_kernel(BATCH, HIST, V, D, DPAD):
    assert HIST == S1 + S2
    b_per_w = BATCH // NW
    n_sc = b_per_w // SCB
    assert n_sc % 2 == 0
    idx_per_sc = SCB * HIST

    mesh = plsc.VectorSubcoreMesh(core_axis_name="c", subcore_axis_name="s")

    @functools.partial(
        pl.kernel,
        mesh=mesh,
        out_type=jax.ShapeDtypeStruct((BATCH, HIST, D), jnp.float32),
        scratch_types=[
            pltpu.VMEM((idx_per_sc,), jnp.int32),
            pltpu.VMEM((idx_per_sc,), jnp.int32),
            pltpu.VMEM((HIST, DPAD), jnp.float32),
            pltpu.VMEM((HIST, DPAD), jnp.float32),
            pltpu.VMEM((HIST, D), jnp.float32),
            pltpu.VMEM((HIST, D), jnp.float32),
            pltpu.SemaphoreType.DMA,
            pltpu.SemaphoreType.DMA,
            pltpu.SemaphoreType.DMA,
            pltpu.SemaphoreType.DMA,
            pltpu.SemaphoreType.DMA,
            pltpu.SemaphoreType.DMA,
        ],
    )
    def emb(idx_hbm, table_hbm, out_hbm,
            ix0, ix1, a0, a1, br0, br1, si0, si1, sg0, sg1, so0, so1):
        idx_bufs = (ix0, ix1)
        a_bufs = (a0, a1)
        bridges = (br0, br1)
        sem_i = (si0, si1)
        sem_g = (sg0, sg1)
        sem_o = (so0, so1)
        wid = lax.axis_index("s") * NC + lax.axis_index("c")
        b0_w = wid * b_per_w
        i0_w = b0_w * HIST

        def fire_idx(sc, p):
            pltpu.async_copy(
                idx_hbm.at[pl.ds(i0_w + sc * idx_per_sc, idx_per_sc)],
                idx_bufs[p], sem_i[p],
            )

        def wait_idx(p):
            pltpu.make_async_copy(
                idx_hbm.at[pl.ds(0, idx_per_sc)], idx_bufs[p], sem_i[p]
            ).wait()

        def fire_gathers(u, ip, ap):
            pltpu.async_copy(
                table_hbm.at[idx_bufs[ip].at[pl.ds(u * HIST, S1)]],
                a_bufs[ap].at[pl.ds(0, S1)], sem_g[ap],
            )
            pltpu.async_copy(
                table_hbm.at[idx_bufs[ip].at[pl.ds(u * HIST + S1, S2)]],
                a_bufs[ap].at[pl.ds(S1, S2)], sem_g[ap],
            )

        def wait_gathers(p):
            pltpu.make_async_copy(
                table_hbm.at[pl.ds(0, HIST)], a_bufs[p], sem_g[p]
            ).wait()

        def fire_out(b, p):
            pltpu.async_copy(bridges[p], out_hbm.at[b], sem_o[p])

        def wait_out(p):
            pltpu.make_async_copy(
                bridges[p], out_hbm.at[0], sem_o[p]
            ).wait()

        def compact(p):
            av = a_bufs[p]
            bv = bridges[p]

            def blk(_, r0):
                vals = []
                for rr in range(4):
                    for o in (0, 16, 32, D - 16):
                        vals.append((rr, o, av[r0 + rr, pl.ds(o, 16)]))
                for rr, o, v in vals:
                    bv[r0 + rr, pl.ds(o, 16)] = v
                return r0 + 4

            lax.fori_loop(0, HIST // 4, blk, 0)

        def phase(t, delta, u):
            sc = t * 2 + delta
            bl = sc * SCB + u
            ap = u % 2
            an = (u + 1) % 2
            ip = delta if u < SCB - 1 else 1 - delta
            un = (u + 1) % SCB

            if u == SCB - 1:
                @pl.when(bl + 1 < b_per_w)
                def _():
                    wait_idx(1 - delta)

            @pl.when(bl + 1 < b_per_w)
            def _():
                fire_gathers(un, ip, an)

            wait_gathers(ap)

            @pl.when(bl >= 2)
            def _():
                wait_out(ap)

            compact(ap)
            fire_out(b0_w + bl, ap)

            if u == SCB - 1:
                @pl.when(sc + 2 < n_sc)
                def _():
                    fire_idx(sc + 2, delta)

        fire_idx(0, 0)
        wait_idx(0)
        fire_idx(1, 1)
        fire_gathers(0, 0, 0)

        def pair_body(t, _):
            for delta in (0, 1):
                for u in range(SCB):
                    phase(t, delta, u)
            return ()

        lax.fori_loop(0, n_sc // 2, pair_body, ())

        wait_out(0)
        wait_out(1)

    return emb


def kernel(indices, table):
    BATCH, HIST = indices.shape
    V, D = table.shape
    DPAD = 128
    idx_flat = indices.reshape(BATCH * HIST).astype(jnp.int32)
    table_p = jnp.pad(table, ((0, 0), (0, DPAD - D)))
    emb = _make_emb_kernel(BATCH, HIST, V, D, DPAD)
    return emb(idx_flat, table_p)


# speedup vs baseline: 1.9199x; 1.0009x over previous
import functools

import jax
import jax.numpy as jnp
from jax import lax
from jax.experimental import pallas as pl
from jax.experimental.pallas import tpu as pltpu
from jax.experimental.pallas import tpu_sc as plsc

NC = 2
NS = 16
NW = NC * NS

SCB = 16
S1, S2 = 104, 96


def _make_emb_kernel(BATCH, HIST, V, D, DPAD):
    assert HIST == S1 + S2
    b_per_w = BATCH // NW
    n_sc = b_per_w // SCB
    assert n_sc % 2 == 0
    idx_per_sc = SCB * HIST

    mesh = plsc.VectorSubcoreMesh(core_axis_name="c", subcore_axis_name="s")

    @functools.partial(
        pl.kernel,
        mesh=mesh,
        out_type=jax.ShapeDtypeStruct((BATCH, HIST, D), jnp.float32),
        scratch_types=[
            pltpu.VMEM((idx_per_sc,), jnp.int32),
            pltpu.VMEM((idx_per_sc,), jnp.int32),
            pltpu.VMEM((HIST, DPAD), jnp.float32),
            pltpu.VMEM((HIST, DPAD), jnp.float32),
            pltpu.VMEM((HIST, D), jnp.float32),
            pltpu.VMEM((HIST, D), jnp.float32),
            pltpu.SemaphoreType.DMA,
            pltpu.SemaphoreType.DMA,
            pltpu.SemaphoreType.DMA,
            pltpu.SemaphoreType.DMA,
            pltpu.SemaphoreType.DMA,
            pltpu.SemaphoreType.DMA,
        ],
        compiler_params=pltpu.CompilerParams(needs_layout_passes=True),
    )
    def emb(idx_hbm, table_hbm, out_hbm,
            ix0, ix1, a0, a1, br0, br1, si0, si1, sg0, sg1, so0, so1):
        idx_bufs = (ix0, ix1)
        a_bufs = (a0, a1)
        bridges = (br0, br1)
        sem_i = (si0, si1)
        sem_g = (sg0, sg1)
        sem_o = (so0, so1)
        wid = lax.axis_index("s") * NC + lax.axis_index("c")
        b0_w = wid * b_per_w
        i0_w = b0_w * HIST

        def fire_idx(sc, p):
            pltpu.async_copy(
                idx_hbm.at[pl.ds(i0_w + sc * idx_per_sc, idx_per_sc)],
                idx_bufs[p], sem_i[p],
            )

        def wait_idx(p):
            pltpu.make_async_copy(
                idx_hbm.at[pl.ds(0, idx_per_sc)], idx_bufs[p], sem_i[p]
            ).wait()

        def fire_gathers(u, ip, ap):
            pltpu.async_copy(
                table_hbm.at[idx_bufs[ip].at[pl.ds(u * HIST, S1)]],
                a_bufs[ap].at[pl.ds(0, S1)], sem_g[ap],
            )
            pltpu.async_copy(
                table_hbm.at[idx_bufs[ip].at[pl.ds(u * HIST + S1, S2)]],
                a_bufs[ap].at[pl.ds(S1, S2)], sem_g[ap],
            )

        def wait_gathers(p):
            pltpu.make_async_copy(
                table_hbm.at[pl.ds(0, HIST)], a_bufs[p], sem_g[p]
            ).wait()

        def fire_out(b, p):
            pltpu.async_copy(bridges[p], out_hbm.at[b], sem_o[p])

        def wait_out(p):
            pltpu.make_async_copy(
                bridges[p], out_hbm.at[0], sem_o[p]
            ).wait()

        def compact(p):
            av = a_bufs[p]
            bv = bridges[p]

            def blk(_, r0):
                vals = []
                for rr in range(4):
                    for o in (0, 16, 32, D - 16):
                        vals.append((rr, o, av[r0 + rr, pl.ds(o, 16)]))
                for rr, o, v in vals:
                    bv[r0 + rr, pl.ds(o, 16)] = v
                return r0 + 4

            lax.fori_loop(0, HIST // 4, blk, 0)

        def phase(t, delta, u):
            sc = t * 2 + delta
            bl = sc * SCB + u
            ap = u % 2
            an = (u + 1) % 2
            ip = delta if u < SCB - 1 else 1 - delta
            un = (u + 1) % SCB

            if u == SCB - 1:
                @pl.when(bl + 1 < b_per_w)
                def _():
                    wait_idx(1 - delta)

            @pl.when(bl + 1 < b_per_w)
            def _():
                fire_gathers(un, ip, an)

            wait_gathers(ap)

            @pl.when(bl >= 2)
            def _():
                wait_out(ap)

            compact(ap)
            fire_out(b0_w + bl, ap)

            if u == SCB - 1:
                @pl.when(sc + 2 < n_sc)
                def _():
                    fire_idx(sc + 2, delta)

        fire_idx(0, 0)
        wait_idx(0)
        fire_idx(1, 1)
        fire_gathers(0, 0, 0)

        def pair_body(t, _):
            for delta in (0, 1):
                for u in range(SCB):
                    phase(t, delta, u)
            return ()

        lax.fori_loop(0, n_sc // 2, pair_body, ())

        wait_out(0)
        wait_out(1)

    return emb


def kernel(indices, table):
    BATCH, HIST = indices.shape
    V, D = table.shape
    DPAD = 128
    idx_flat = indices.reshape(BATCH * HIST).astype(jnp.int32)
    table_p = jnp.pad(table, ((0, 0), (0, DPAD - D)))
    emb = _make_emb_kernel(BATCH, HIST, V, D, DPAD)
    return emb(idx_flat, table_p)
